# baseline (speedup 1.0000x reference)
import functools

import jax
import jax.numpy as jnp
import numpy as np
from jax import lax
from jax.experimental import pallas as pl
from jax.experimental.pallas import tpu as pltpu
from jax.experimental.pallas import tpu_sc as plsc

_N = 36
_PAD = 48
_L = 16

_TWO_PI = float(np.pi * 2)
_HALF_PI = float(np.pi / 2)
_WINDOW = float(np.pi / 3 / 2)

_MESH = plsc.VectorSubcoreMesh(
    core_axis_name="c", subcore_axis_name="s", num_cores=2, num_subcores=16
)


def _fmod_pos(x, m):
    r = lax.rem(x, jnp.float32(m))
    return jnp.where(r < 0, r + jnp.float32(m), r)


@functools.partial(
    pl.kernel,
    mesh=_MESH,
    compiler_params=pltpu.CompilerParams(needs_layout_passes=False),
    out_type=jax.ShapeDtypeStruct((_L,), jnp.float32),
    scratch_types=[
        pltpu.VMEM((_PAD,), jnp.float32),
        pltpu.VMEM((_PAD,), jnp.float32),
        pltpu.VMEM((8 * _PAD,), jnp.float32),
        pltpu.VMEM((_PAD,), jnp.float32),
        pltpu.VMEM((_PAD,), jnp.float32),
        pltpu.VMEM((_PAD,), jnp.float32),
        pltpu.VMEM((_PAD,), jnp.float32),
        pltpu.VMEM((_PAD,), jnp.float32),
        pltpu.VMEM((_L,), jnp.float32),
    ],
)
def _plant_loss_sc(y0_hbm, y1_hbm, preds_hbm, out_hbm,
                   y0_in, y1_in, preds_v, y0m_r, y1m_r, keys_r,
                   s0_r, s1_r, outv):
    wid = lax.axis_index("s") * 2 + lax.axis_index("c")

    @pl.when(wid == 0)
    def _():
        pltpu.sync_copy(y0_hbm, y0_in)
        pltpu.sync_copy(y1_hbm, y1_in)
        pltpu.sync_copy(preds_hbm, preds_v)

        lane = lax.iota(jnp.int32, _L)
        lane0 = lane == 0
        acc = jnp.zeros((_L,), jnp.float32)

        for i in range(4):
            keys = []
            for s in range(3):
                sl = pl.ds(s * _L, _L)
                ang = y1_in[sl]
                a0 = y0_in[sl]
                gidx = lane + (s * _L)
                valid = gidx < _N
                t = ang - jnp.float32(_HALF_PI * i)
                factor = (_fmod_pos(t, _TWO_PI) < jnp.float32(_WINDOW)) & valid
                y1m = jnp.where(factor, ang, jnp.float32(0.0))
                y0m = jnp.where(factor, a0, jnp.float32(0.0))
                key = jnp.where(valid, y1m, jnp.float32(np.inf))
                y0m_r[sl] = y0m
                y1m_r[sl] = y1m
                keys_r[sl] = key
                keys.append(key)

            for j in range(_N):
                jd = jnp.full((_L,), j, jnp.int32)
                bj = plsc.load_gather(keys_r, [jd])
                v0j = plsc.load_gather(y0m_r, [jd])
                v1j = plsc.load_gather(y1m_r, [jd])
                rank = jnp.zeros((_L,), jnp.int32)
                for s in range(3):
                    gidx = lane + (s * _L)
                    cond = (keys[s] < bj) | ((keys[s] == bj) & (gidx < j))
                    rank = rank + plsc.all_reduce_population_count(cond)
                plsc.store_scatter(s0_r, [rank], v0j, mask=lane0)
                plsc.store_scatter(s1_r, [rank], v1j, mask=lane0)

            for s in range(3):
                sl = pl.ds(s * _L, _L)
                gidx = lane + (s * _L)
                valid = gidx < _N
                d0 = _fmod_pos(s0_r[sl], 6.28) / jnp.float32(6.28) \
                    - preds_v[pl.ds((i * 2) * _PAD + s * _L, _L)]
                d1 = _fmod_pos(s1_r[sl], 6.28) / jnp.float32(6.28) \
                    - preds_v[pl.ds((i * 2 + 1) * _PAD + s * _L, _L)]
                acc = acc + jnp.where(valid, d0 * d0 + d1 * d1,
                                      jnp.float32(0.0))

        total = jnp.sum(acc)
        outv[...] = jnp.full((_L,), total) / jnp.float32(_N * 2 * 4)
        pltpu.sync_copy(outv, out_hbm)


def kernel(y, preds0, preds1, preds2, preds3):
    y0 = jnp.pad(y[0, :_N, 0], (0, _PAD - _N))
    y1 = jnp.pad(y[0, :_N, 1], (0, _PAD - _N))
    preds = jnp.stack([preds0, preds1, preds2, preds3])
    preds = jnp.transpose(preds, (0, 2, 1))
    preds = jnp.pad(preds, ((0, 0), (0, 0), (0, _PAD - _N))).reshape(8 * _PAD)
    out = _plant_loss_sc(y0, y1, preds)
    return out[0]


# speedup vs baseline: 1.1837x; 1.1837x over previous
import functools

import jax
import jax.numpy as jnp
import numpy as np
from jax import lax
from jax.experimental import pallas as pl
from jax.experimental.pallas import tpu as pltpu
from jax.experimental.pallas import tpu_sc as plsc

_N = 36
_L = 16

_TWO_PI = float(np.pi * 2)
_HALF_PI = float(np.pi / 2)
_WINDOW = float(np.pi / 3 / 2)

_MESH = plsc.VectorSubcoreMesh(
    core_axis_name="c", subcore_axis_name="s", num_cores=1, num_subcores=16
)


def _fmod_pos(x, m):
    r = lax.rem(x, jnp.float32(m))
    return jnp.where(r < 0, r + jnp.float32(m), r)


def _merge(ak, av, bk, bv):
    rk = lax.rev(bk, (0,))
    rv = lax.rev(bv, (0,))
    m = ak <= rk
    lo = plsc.sort_key_val(jnp.where(m, ak, rk), jnp.where(m, av, rv))
    hi = plsc.sort_key_val(jnp.where(m, rk, ak), jnp.where(m, rv, av))
    return lo[0], lo[1], hi[0], hi[1]


@functools.partial(
    pl.kernel,
    mesh=_MESH,
    compiler_params=pltpu.CompilerParams(needs_layout_passes=False),
    out_type=jax.ShapeDtypeStruct((_L,), jnp.float32),
    scratch_types=[
        pltpu.VMEM((128,), jnp.float32),
        pltpu.VMEM((4 * 72,), jnp.float32),
        pltpu.VMEM((_L,), jnp.float32),
    ],
)
def _plant_loss_sc(y_hbm, p0_hbm, p1_hbm, p2_hbm, p3_hbm, out_hbm,
                   y_v, preds_v, outv):
    wid = lax.axis_index("s")

    @pl.when(wid == 0)
    def _():
        pltpu.sync_copy(y_hbm, y_v)
        pltpu.sync_copy(p0_hbm, preds_v.at[pl.ds(0, 72)])
        pltpu.sync_copy(p1_hbm, preds_v.at[pl.ds(72, 72)])
        pltpu.sync_copy(p2_hbm, preds_v.at[pl.ds(144, 72)])
        pltpu.sync_copy(p3_hbm, preds_v.at[pl.ds(216, 72)])

        lane = lax.iota(jnp.int32, _L)
        acc = jnp.zeros((_L,), jnp.float32)

        for i in range(4):
            K, V = [], []
            for s in range(3):
                gidx = lane + (s * _L)
                valid = gidx < _N
                a0 = plsc.load_gather(y_v, [gidx * 2])
                ang = plsc.load_gather(y_v, [gidx * 2 + 1])
                t = ang - jnp.float32(_HALF_PI * i)
                factor = (_fmod_pos(t, _TWO_PI) < jnp.float32(_WINDOW)) & valid
                y1m = jnp.where(factor, ang, jnp.float32(0.0))
                y0m = jnp.where(factor, a0, jnp.float32(0.0))
                key = jnp.where(valid, y1m, jnp.float32(np.inf))
                sk, sv = plsc.sort_key_val(key, y0m)
                K.append(sk)
                V.append(sv)

            K[0], V[0], K[1], V[1] = _merge(K[0], V[0], K[1], V[1])
            K[1], V[1], K[2], V[2] = _merge(K[1], V[1], K[2], V[2])
            K[0], V[0], K[1], V[1] = _merge(K[0], V[0], K[1], V[1])

            for s in range(3):
                gidx = lane + (s * _L)
                valid = gidx < _N
                pidx = jnp.where(valid, gidx * 2, 0) + (i * 72)
                p0 = plsc.load_gather(preds_v, [pidx])
                p1 = plsc.load_gather(preds_v, [pidx + 1])
                d0 = _fmod_pos(V[s], 6.28) / jnp.float32(6.28) - p0
                d1 = _fmod_pos(K[s], 6.28) / jnp.float32(6.28) - p1
                acc = acc + jnp.where(valid, d0 * d0 + d1 * d1,
                                      jnp.float32(0.0))

        total = jnp.sum(acc)
        outv[...] = jnp.full((_L,), total) / jnp.float32(_N * 2 * 4)
        pltpu.sync_copy(outv, out_hbm)


def kernel(y, preds0, preds1, preds2, preds3):
    out = _plant_loss_sc(
        y.reshape(128),
        preds0.reshape(72), preds1.reshape(72),
        preds2.reshape(72), preds3.reshape(72),
    )
    return out[0]
